# Initial kernel scaffold; baseline (speedup 1.0000x reference)
import jax
import jax.numpy as jnp
from jax.experimental import pallas as pl


def kernel(x, edge_index, edge_attr, edge_type, Wrel, Wself, Wedge, b, W1, b1, W2, b2, gamma, beta):
    raise NotImplementedError("write your pallas kernel here")


# speedup vs baseline: 9.9728x; 9.9728x over previous
import functools

import jax
import jax.numpy as jnp
from jax import lax
from jax.experimental import pallas as pl
from jax.experimental.pallas import tpu as pltpu
from jax.experimental.pallas import tpu_sc as plsc

_NC = 2
_NS = 16
_NW = _NC * _NS
_CH = 128


def _ceil_to(a, m):
  return (a + m - 1) // m * m


def _build_attr_pass(E_pad, N_pad, nch):
  epw = E_pad // _NW
  rpt = N_pad // _NS

  @functools.partial(
      pl.kernel,
      out_type=jax.ShapeDtypeStruct((_NC, N_pad, 128), jnp.float32),
      mesh=plsc.VectorSubcoreMesh(core_axis_name="c", subcore_axis_name="s", num_cores=_NC, num_subcores=_NS),
      scratch_types=[
          pltpu.VMEM((_CH,), jnp.int32),
          pltpu.VMEM((_CH, 128), jnp.float32),
          pltpu.VMEM_SHARED((N_pad, 128), jnp.float32),
      ],
  )
  def attr_pass(attr_hbm, didx_hbm, zeros_hbm, out_hbm, didx_v, rows_v, acc_sh):
    c = lax.axis_index("c")
    s = lax.axis_index("s")
    wid = s * _NC + c
    pltpu.sync_copy(zeros_hbm.at[pl.ds(s * rpt, rpt)],
                    acc_sh.at[pl.ds(s * rpt, rpt)])
    plsc.subcore_barrier()

    def body(i, carry):
      base = wid * epw + i * _CH
      pltpu.sync_copy(didx_hbm.at[pl.ds(base, _CH)], didx_v)
      pltpu.sync_copy(attr_hbm.at[pl.ds(base, _CH)], rows_v)
      pltpu.sync_copy(rows_v, acc_sh.at[didx_v], add=True)
      return carry

    lax.fori_loop(0, nch, body, 0)
    plsc.subcore_barrier()
    pltpu.sync_copy(acc_sh.at[pl.ds(s * rpt, rpt)],
                    out_hbm.at[c, pl.ds(s * rpt, rpt)])

  return attr_pass


def _build_edge_pass(RN, D, E_pad, N_pad, nch):
  epw = E_pad // _NW
  rpt = N_pad // _NS

  @functools.partial(
      pl.kernel,
      out_type=jax.ShapeDtypeStruct((_NC, N_pad, D), jnp.float32),
      mesh=plsc.VectorSubcoreMesh(core_axis_name="c", subcore_axis_name="s", num_cores=_NC, num_subcores=_NS),
      scratch_types=[
          pltpu.VMEM((_CH,), jnp.int32),
          pltpu.VMEM((_CH,), jnp.int32),
          pltpu.VMEM((_CH, 128), jnp.float32),
          pltpu.VMEM_SHARED((N_pad, 128), jnp.float32),
          pltpu.SemaphoreType.DMA,
      ],
  )
  def edge_pass(xr_hbm, gidx_hbm, didx_hbm, zeros_hbm, out_hbm,
                gidx_v, didx_v, rows_v, acc_sh, sem):
    c = lax.axis_index("c")
    s = lax.axis_index("s")
    wid = s * _NC + c
    pltpu.sync_copy(zeros_hbm.at[pl.ds(s * rpt, rpt)],
                    acc_sh.at[pl.ds(s * rpt, rpt)])
    plsc.subcore_barrier()

    def body(i, carry):
      base = wid * epw + i * _CH
      pltpu.sync_copy(gidx_hbm.at[pl.ds(base, _CH)], gidx_v)
      pltpu.sync_copy(didx_hbm.at[pl.ds(base, _CH)], didx_v)
      pltpu.async_copy(xr_hbm.at[gidx_v], rows_v, sem).wait()
      pltpu.sync_copy(rows_v, acc_sh.at[didx_v], add=True)
      return carry

    lax.fori_loop(0, nch, body, 0)
    plsc.subcore_barrier()
    pltpu.sync_copy(acc_sh.at[pl.ds(s * rpt, rpt)],
                    out_hbm.at[c, pl.ds(s * rpt, rpt)])

  return edge_pass


def _rel0_body(R, h_ref, w_ref, xr_ref):
  h = h_ref[...]
  for r in range(R):
    xr_ref[r] = jnp.dot(h, w_ref[r], preferred_element_type=jnp.float32)


def _post_body(EA, B, p_ref, h_ref, ae_ref, wself_ref, wedge_ref, b_ref,
               w1_ref, b1_ref, w2_ref, b2_ref, hpre_ref, st_ref):
  i = pl.program_id(0)
  ae = ae_ref[0] + ae_ref[1]
  cnt = jnp.maximum(ae[:, EA:EA + 1], 1.0)
  psum = p_ref[0] + p_ref[1]
  agg = (psum + jnp.dot(ae[:, :EA], wedge_ref[...],
                        preferred_element_type=jnp.float32,
                  precision=lax.Precision.HIGHEST)) / cnt
  t = agg + jnp.dot(h_ref[...], wself_ref[...],
                    preferred_element_type=jnp.float32) + b_ref[...]
  u = jnp.maximum(jnp.dot(t, w1_ref[...],
                          preferred_element_type=jnp.float32) + b1_ref[...], 0.0)
  hp = jnp.dot(u, w2_ref[...], preferred_element_type=jnp.float32) + b2_ref[...]
  hpre_ref[...] = hp
  ones8 = jnp.ones((8, B), jnp.float32)
  ssum = jnp.dot(ones8, hp, preferred_element_type=jnp.float32,
                  precision=lax.Precision.HIGHEST)
  ssq = jnp.dot(ones8, hp * hp, preferred_element_type=jnp.float32,
                  precision=lax.Precision.HIGHEST)

  @pl.when(i == 0)
  def _():
    st_ref[0] = ssum
    st_ref[1] = ssq

  @pl.when(i > 0)
  def _():
    st_ref[0] += ssum
    st_ref[1] += ssq


def _relbn_body(R, N, hp_ref, st_ref, g_ref, be_ref, w_ref, h_ref, xr_ref):
  mu = st_ref[0, 0:1, :] * (1.0 / N)
  ex2 = st_ref[1, 0:1, :] * (1.0 / N)
  var = ex2 - mu * mu
  rstd = lax.rsqrt(var + 1e-5)
  hb = (hp_ref[...] - mu) * (rstd * g_ref[...]) + be_ref[...]
  hb = jnp.maximum(hb, 0.0)
  h_ref[...] = hb
  for r in range(R):
    xr_ref[r] = jnp.dot(hb, w_ref[r], preferred_element_type=jnp.float32)


def _bnfin_body(N, hp_ref, st_ref, g_ref, be_ref, out_ref):
  mu = st_ref[0, 0:1, :] * (1.0 / N)
  ex2 = st_ref[1, 0:1, :] * (1.0 / N)
  var = ex2 - mu * mu
  rstd = lax.rsqrt(var + 1e-5)
  out_ref[...] = (hp_ref[...] - mu) * (rstd * g_ref[...]) + be_ref[...]


def kernel(x, edge_index, edge_attr, edge_type, Wrel, Wself, Wedge, b,
           W1, b1, W2, b2, gamma, beta):
  N, D = x.shape
  E = edge_index.shape[1]
  EA = edge_attr.shape[1]
  L, R = Wrel.shape[0], Wrel.shape[1]

  src = edge_index[0]
  dst = edge_index[1]
  gidx = edge_type * N + src

  E_pad = _ceil_to(E, _NW * _CH)
  nch = E_pad // (_NW * _CH)
  N_pad = _ceil_to(N + _NS, _NS * 8)
  npad_e = E_pad - E
  pad_dst = N + (jnp.arange(npad_e, dtype=jnp.int32) % _NS)
  gidx_p = jnp.concatenate([gidx, jnp.zeros((npad_e,), jnp.int32)])
  didx_p = jnp.concatenate([dst, pad_dst])
  attr_bf = edge_attr.astype(jnp.bfloat16).astype(jnp.float32)
  wedge_bf = Wedge.astype(jnp.bfloat16).astype(jnp.float32)
  attr_ext = jnp.zeros((E_pad, 128), jnp.float32)
  attr_ext = attr_ext.at[:E, :EA].set(attr_bf)
  attr_ext = attr_ext.at[:E, EA].set(1.0)

  zeros_d = jnp.zeros((N_pad, D), jnp.float32)

  b2d = b.reshape(L, 1, D)
  b12d = b1.reshape(L, 1, 2 * D)
  b22d = b2.reshape(L, 1, D)
  g2d = gamma.reshape(L, 1, D)
  be2d = beta.reshape(L, 1, D)

  B = 1000
  nb = N // B

  attr_pass = _build_attr_pass(E_pad, N_pad, nch)
  edge_pass = _build_edge_pass(R * N, D, E_pad, N_pad, nch)

  ae_part = attr_pass(attr_ext, didx_p, zeros_d)

  rel0 = pl.pallas_call(
      functools.partial(_rel0_body, R),
      grid=(nb,),
      in_specs=[
          pl.BlockSpec((B, D), lambda i: (i, 0)),
          pl.BlockSpec((R, D, D), lambda i: (0, 0, 0)),
      ],
      out_specs=pl.BlockSpec((R, B, D), lambda i: (0, i, 0)),
      out_shape=jax.ShapeDtypeStruct((R, N, D), jnp.float32),
  )

  post = pl.pallas_call(
      functools.partial(_post_body, EA, B),
      grid=(nb,),
      in_specs=[
          pl.BlockSpec((_NC, B, D), lambda i: (0, i, 0)),
          pl.BlockSpec((B, D), lambda i: (i, 0)),
          pl.BlockSpec((_NC, B, 128), lambda i: (0, i, 0)),
          pl.BlockSpec((D, D), lambda i: (0, 0)),
          pl.BlockSpec((EA, D), lambda i: (0, 0)),
          pl.BlockSpec((1, D), lambda i: (0, 0)),
          pl.BlockSpec((D, 2 * D), lambda i: (0, 0)),
          pl.BlockSpec((1, 2 * D), lambda i: (0, 0)),
          pl.BlockSpec((2 * D, D), lambda i: (0, 0)),
          pl.BlockSpec((1, D), lambda i: (0, 0)),
      ],
      out_specs=[
          pl.BlockSpec((B, D), lambda i: (i, 0)),
          pl.BlockSpec((2, 8, D), lambda i: (0, 0, 0)),
      ],
      out_shape=[
          jax.ShapeDtypeStruct((N, D), jnp.float32),
          jax.ShapeDtypeStruct((2, 8, D), jnp.float32),
      ],
  )

  relbn = pl.pallas_call(
      functools.partial(_relbn_body, R, N),
      grid=(nb,),
      in_specs=[
          pl.BlockSpec((B, D), lambda i: (i, 0)),
          pl.BlockSpec((2, 8, D), lambda i: (0, 0, 0)),
          pl.BlockSpec((1, D), lambda i: (0, 0)),
          pl.BlockSpec((1, D), lambda i: (0, 0)),
          pl.BlockSpec((R, D, D), lambda i: (0, 0, 0)),
      ],
      out_specs=[
          pl.BlockSpec((B, D), lambda i: (i, 0)),
          pl.BlockSpec((R, B, D), lambda i: (0, i, 0)),
      ],
      out_shape=[
          jax.ShapeDtypeStruct((N, D), jnp.float32),
          jax.ShapeDtypeStruct((R, N, D), jnp.float32),
      ],
  )

  bnfin = pl.pallas_call(
      functools.partial(_bnfin_body, N),
      grid=(nb,),
      in_specs=[
          pl.BlockSpec((B, D), lambda i: (i, 0)),
          pl.BlockSpec((2, 8, D), lambda i: (0, 0, 0)),
          pl.BlockSpec((1, D), lambda i: (0, 0)),
          pl.BlockSpec((1, D), lambda i: (0, 0)),
      ],
      out_specs=pl.BlockSpec((B, D), lambda i: (i, 0)),
      out_shape=jax.ShapeDtypeStruct((N, D), jnp.float32),
  )

  h = x
  xr = rel0(x, Wrel[0])
  for l in range(L):
    P = edge_pass(xr.reshape(R * N, D), gidx_p, didx_p, zeros_d)
    h_pre, stats = post(P, h, ae_part, Wself[l], wedge_bf[l], b2d[l],
                        W1[l], b12d[l], W2[l], b22d[l])
    if l < L - 1:
      h, xr = relbn(h_pre, stats, g2d[l], be2d[l], Wrel[l + 1])
    else:
      return bnfin(h_pre, stats, g2d[l], be2d[l])
